# baseline (speedup 1.0000x reference)
import functools

import jax
import jax.numpy as jnp
from jax import lax
from jax.experimental import pallas as pl
from jax.experimental.pallas import tpu as pltpu
from jax.experimental.pallas import tpu_sc as plsc

N_NODES = 10000
N_EDGES = 160000
NFEAT = 256
NHID = 256
NCLASS = 64
NCLS_P = 128

NC = 2
NS = 16

ROW_BLK = 1000
N_ROW_BLK = N_NODES // ROW_BLK

DRCH = 80
NDRCH = N_NODES // DRCH
DR_PER_TILE = -(-NDRCH // NS)

CHW1 = 80
CHW2 = 100
IB = 25
NB1 = N_EDGES // NS // (IB * CHW1)
NB2 = N_EDGES // (NC * NS) // (IB * CHW2)
D1 = 4
D2 = 4


@functools.lru_cache(maxsize=None)
def _mesh():
    return plsc.VectorSubcoreMesh(
        core_axis_name="c", subcore_axis_name="s", num_cores=NC, num_subcores=NS
    )


def _mm1_body(x_ref, w_ref, o_ref):
    o_ref[...] = jnp.dot(x_ref[...], w_ref[...],
                         preferred_element_type=jnp.float32)


def _mm1(x, W1):
    return pl.pallas_call(
        _mm1_body,
        grid=(NC, N_ROW_BLK),
        in_specs=[
            pl.BlockSpec((ROW_BLK, NFEAT), lambda h, i: (i, 0)),
            pl.BlockSpec((NFEAT, 128), lambda h, i: (0, h)),
        ],
        out_specs=pl.BlockSpec((ROW_BLK, 128),
                               lambda h, i: (h * N_ROW_BLK + i, 0)),
        out_shape=jax.ShapeDtypeStruct((NC * N_NODES, 128), jnp.float32),
    )(x, W1)


def _zero_acc(zeros_hbm, acc, s):
    for i in range(DR_PER_TILE):
        ch = s + NS * i

        @pl.when(ch < NDRCH)
        def _():
            pltpu.sync_copy(zeros_hbm, acc.at[pl.ds(ch * DRCH, DRCH)])


@functools.lru_cache(maxsize=None)
def _spmm1_kernel():
    @functools.partial(
        pl.kernel,
        out_type=jax.ShapeDtypeStruct((NC * N_NODES, 128), jnp.float32),
        mesh=_mesh(),
        scratch_types=[
            pltpu.VMEM((IB, CHW1), jnp.int32),
            pltpu.VMEM((IB, CHW1), jnp.int32),
            [pltpu.VMEM((CHW1, 128), jnp.float32) for _ in range(D1)],
            pltpu.VMEM_SHARED((N_NODES, 128), jnp.float32),
            [pltpu.SemaphoreType.DMA for _ in range(D1)],
        ],
    )
    def spmm1(sup_hbm, src_hbm, dst_hbm, zeros_hbm, h_hbm,
              isrc_v, idst_v, bufs, acc, sems):
        c = lax.axis_index("c")
        s = lax.axis_index("s")

        _zero_acc(zeros_hbm, acc, s)
        plsc.subcore_barrier()

        half = sup_hbm.at[pl.ds(c * N_NODES, N_NODES)]

        def gather(j, k):
            pltpu.async_copy(half.at[isrc_v.at[j]], bufs[k], sems[k])

        def wait_gather(k):
            pltpu.make_async_copy(
                half.at[isrc_v.at[0]], bufs[k], sems[k]).wait()

        def block_body(b, carry):
            pltpu.sync_copy(src_hbm.at[s, b], isrc_v)
            pltpu.sync_copy(dst_hbm.at[s, b], idst_v)
            for k in range(D1 - 1):
                gather(k, k)
            for j in range(IB):
                wait_gather(j % D1)
                if j + D1 - 1 < IB:
                    gather(j + D1 - 1, (j + D1 - 1) % D1)
                pltpu.sync_copy(bufs[j % D1], acc.at[idst_v.at[j]], add=True)
            return carry

        lax.fori_loop(0, NB1, block_body, 0)

        plsc.subcore_barrier()

        for i in range(DR_PER_TILE):
            ch = s + NS * i

            @pl.when(ch < NDRCH)
            def _():
                r0 = ch * DRCH
                pltpu.sync_copy(acc.at[pl.ds(r0, DRCH)],
                                h_hbm.at[pl.ds(c * N_NODES + r0, DRCH)])

    return spmm1


def _mm2_body(h0_ref, h1_ref, b1a_ref, b1b_ref, w2_ref, o_ref):
    h0 = jnp.maximum(h0_ref[...] + b1a_ref[0:1, :], 0.0)
    h1 = jnp.maximum(h1_ref[...] + b1b_ref[0:1, :], 0.0)
    a = jnp.dot(h0, w2_ref[:128, :], preferred_element_type=jnp.float32)
    b = jnp.dot(h1, w2_ref[128:, :], preferred_element_type=jnp.float32)
    o_ref[...] = a + b


NCLS = NCLASS


def _mm2(h_all, b1a, b1b, W2):
    return pl.pallas_call(
        _mm2_body,
        grid=(N_ROW_BLK,),
        in_specs=[
            pl.BlockSpec((ROW_BLK, 128), lambda i: (i, 0)),
            pl.BlockSpec((ROW_BLK, 128), lambda i: (N_ROW_BLK + i, 0)),
            pl.BlockSpec((8, 128), lambda i: (0, 0)),
            pl.BlockSpec((8, 128), lambda i: (0, 0)),
            pl.BlockSpec((NHID, NCLS), lambda i: (0, 0)),
        ],
        out_specs=pl.BlockSpec((ROW_BLK, NCLS), lambda i: (i, 0)),
        out_shape=jax.ShapeDtypeStruct((N_NODES, NCLS), jnp.float32),
    )(h_all, h_all, b1a, b1b, W2)


@functools.lru_cache(maxsize=None)
def _spmm2_kernel():
    @functools.partial(
        pl.kernel,
        out_type=jax.ShapeDtypeStruct((NC * N_NODES, NCLS), jnp.float32),
        mesh=_mesh(),
        scratch_types=[
            pltpu.VMEM((IB, CHW2), jnp.int32),
            pltpu.VMEM((IB, CHW2), jnp.int32),
            [pltpu.VMEM((CHW2, NCLS), jnp.float32) for _ in range(D2)],
            pltpu.VMEM_SHARED((N_NODES, NCLS), jnp.float32),
            [pltpu.SemaphoreType.DMA for _ in range(D2)],
        ],
        compiler_params=pltpu.CompilerParams(use_tc_tiling_on_sc=False),
    )
    def spmm2(s2_hbm, src_hbm, dst_hbm, zeros_hbm, p_hbm,
              isrc_v, idst_v, bufs, acc, sems):
        c = lax.axis_index("c")
        s = lax.axis_index("s")
        wid = c * NS + s

        _zero_acc(zeros_hbm, acc, s)
        plsc.subcore_barrier()

        def gather(j, k):
            pltpu.async_copy(s2_hbm.at[isrc_v.at[j]], bufs[k], sems[k])

        def wait_gather(k):
            pltpu.make_async_copy(
                s2_hbm.at[isrc_v.at[0]], bufs[k], sems[k]).wait()

        def block_body(b, carry):
            pltpu.sync_copy(src_hbm.at[wid, b], isrc_v)
            pltpu.sync_copy(dst_hbm.at[wid, b], idst_v)
            for k in range(D2 - 1):
                gather(k, k)
            for j in range(IB):
                wait_gather(j % D2)
                if j + D2 - 1 < IB:
                    gather(j + D2 - 1, (j + D2 - 1) % D2)
                pltpu.sync_copy(bufs[j % D2], acc.at[idst_v.at[j]], add=True)
            return carry

        lax.fori_loop(0, NB2, block_body, 0)

        plsc.subcore_barrier()

        for i in range(DR_PER_TILE):
            ch = s + NS * i

            @pl.when(ch < NDRCH)
            def _():
                r0 = ch * DRCH
                pltpu.sync_copy(acc.at[pl.ds(r0, DRCH)],
                                p_hbm.at[pl.ds(c * N_NODES + r0, DRCH)])

    return spmm2


def _comb_body(p0_ref, p1_ref, b2_ref, o_ref):
    o_ref[...] = p0_ref[...] + p1_ref[...] + b2_ref[0:1, :]


def _comb(p_all, b2):
    return pl.pallas_call(
        _comb_body,
        grid=(N_ROW_BLK,),
        in_specs=[
            pl.BlockSpec((ROW_BLK, NCLS), lambda i: (i, 0)),
            pl.BlockSpec((ROW_BLK, NCLS), lambda i: (N_ROW_BLK + i, 0)),
            pl.BlockSpec((8, NCLASS), lambda i: (0, 0)),
        ],
        out_specs=pl.BlockSpec((ROW_BLK, NCLASS), lambda i: (i, 0)),
        out_shape=jax.ShapeDtypeStruct((N_NODES, NCLASS), jnp.float32),
    )(p_all, p_all, b2)


@jax.jit
def kernel(x, adj, W1, b1, W2, b2):
    src = adj[0].astype(jnp.int32)
    dst = adj[1].astype(jnp.int32)
    src1 = src.reshape(NS, NB1, IB, CHW1)
    dst1 = dst.reshape(NS, NB1, IB, CHW1)
    src2 = src.reshape(NC * NS, NB2, IB, CHW2)
    dst2 = dst.reshape(NC * NS, NB2, IB, CHW2)
    zeros1 = jnp.zeros((DRCH, 128), jnp.float32)
    zeros2 = jnp.zeros((DRCH, NCLASS), jnp.float32)
    b1a = jnp.broadcast_to(b1[:128].reshape(1, 128), (8, 128))
    b1b = jnp.broadcast_to(b1[128:].reshape(1, 128), (8, 128))
    b2r = jnp.broadcast_to(b2.reshape(1, NCLASS), (8, NCLASS))

    sup = _mm1(x, W1)
    h_all = _spmm1_kernel()(sup, src1, dst1, zeros1)
    s2 = _mm2(h_all, b1a, b1b, W2)
    p_all = _spmm2_kernel()(s2, src2, dst2, zeros2)
    return _comb(p_all, b2r)


# speedup vs baseline: 1.0944x; 1.0944x over previous
import functools

import jax
import jax.numpy as jnp
from jax import lax
from jax.experimental import pallas as pl
from jax.experimental.pallas import tpu as pltpu
from jax.experimental.pallas import tpu_sc as plsc

N_NODES = 10000
N_EDGES = 160000
NFEAT = 256
NHID = 256
NCLASS = 64
NCLS_P = 128

NC = 2
NS = 16

ROW_BLK = 1000
N_ROW_BLK = N_NODES // ROW_BLK

DRCH = 80
NDRCH = N_NODES // DRCH
DR_PER_TILE = -(-NDRCH // NS)

CHW1 = 80
CHW2 = 100
IB = 25
NB1 = N_EDGES // NS // (IB * CHW1)
NB2 = N_EDGES // (NC * NS) // (IB * CHW2)
D1 = 4
D2 = 4


@functools.lru_cache(maxsize=None)
def _mesh():
    return plsc.VectorSubcoreMesh(
        core_axis_name="c", subcore_axis_name="s", num_cores=NC, num_subcores=NS
    )


def _mm1_body(x_ref, w_ref, o_ref):
    o_ref[...] = jnp.dot(x_ref[...], w_ref[...],
                         preferred_element_type=jnp.float32)


def _mm1_specs():
    return dict(
        grid=(NC, N_ROW_BLK),
        in_specs=[
            pl.BlockSpec((ROW_BLK, NFEAT), lambda h, i: (i, 0)),
            pl.BlockSpec((NFEAT, 128), lambda h, i: (0, h)),
        ],
        out_specs=pl.BlockSpec((ROW_BLK, 128),
                               lambda h, i: (h * N_ROW_BLK + i, 0)),
        out_shape=jax.ShapeDtypeStruct((NC * N_NODES, 128), jnp.float32),
    )


def _mm1(x, W1):
    return pl.pallas_call(
        _mm1_body,
        grid=(NC, N_ROW_BLK),
        in_specs=[
            pl.BlockSpec((ROW_BLK, NFEAT), lambda h, i: (i, 0)),
            pl.BlockSpec((NFEAT, 128), lambda h, i: (0, h)),
        ],
        out_specs=pl.BlockSpec((ROW_BLK, 128),
                               lambda h, i: (h * N_ROW_BLK + i, 0)),
        out_shape=jax.ShapeDtypeStruct((NC * N_NODES, 128), jnp.float32),
    )(x, W1)


def _zero_acc(zeros_hbm, buf_v, acc, s):
    pltpu.sync_copy(zeros_hbm, buf_v)
    for i in range(DR_PER_TILE):
        ch = s + NS * i

        @pl.when(ch < NDRCH)
        def _():
            pltpu.sync_copy(buf_v, acc.at[pl.ds(ch * DRCH, DRCH)])


@functools.lru_cache(maxsize=None)
def _spmm1_kernel():
    @functools.partial(
        pl.kernel,
        out_type=jax.ShapeDtypeStruct((NC * N_NODES, 128), jnp.float32),
        mesh=_mesh(),
        scratch_types=[
            pltpu.VMEM((IB, CHW1), jnp.int32),
            pltpu.VMEM((IB, CHW1), jnp.int32),
            [pltpu.VMEM((CHW1, 128), jnp.float32) for _ in range(D1)],
            pltpu.VMEM_SHARED((N_NODES, 128), jnp.float32),
            [pltpu.SemaphoreType.DMA for _ in range(D1)],
        ],
    )
    def spmm1(sup_hbm, src_hbm, dst_hbm, zeros_hbm, h_hbm,
              isrc_v, idst_v, bufs, acc, sems):
        c = lax.axis_index("c")
        s = lax.axis_index("s")

        _zero_acc(zeros_hbm, bufs[0], acc, s)
        plsc.subcore_barrier()

        half = sup_hbm.at[pl.ds(c * N_NODES, N_NODES)]

        def gather(j, k):
            pltpu.async_copy(half.at[isrc_v.at[j]], bufs[k], sems[k])

        def wait_gather(k):
            pltpu.make_async_copy(
                half.at[isrc_v.at[0]], bufs[k], sems[k]).wait()

        def block_body(b, carry):
            pltpu.sync_copy(src_hbm.at[s, b], isrc_v)
            pltpu.sync_copy(dst_hbm.at[s, b], idst_v)
            for k in range(D1 - 1):
                gather(k, k)
            for j in range(IB):
                wait_gather(j % D1)
                if j + D1 - 1 < IB:
                    gather(j + D1 - 1, (j + D1 - 1) % D1)
                pltpu.sync_copy(bufs[j % D1], acc.at[idst_v.at[j]], add=True)
            return carry

        lax.fori_loop(0, NB1, block_body, 0)

        plsc.subcore_barrier()

        for i in range(DR_PER_TILE):
            ch = s + NS * i

            @pl.when(ch < NDRCH)
            def _():
                r0 = ch * DRCH
                pltpu.sync_copy(acc.at[pl.ds(r0, DRCH)], bufs[0])
                pltpu.sync_copy(bufs[0],
                                h_hbm.at[pl.ds(c * N_NODES + r0, DRCH)])

    return spmm1


def _mm2_body(h0_ref, h1_ref, b1a_ref, b1b_ref, w2_ref, o_ref):
    h0 = jnp.maximum(h0_ref[...] + b1a_ref[0:1, :], 0.0).astype(jnp.bfloat16)
    h1 = jnp.maximum(h1_ref[...] + b1b_ref[0:1, :], 0.0).astype(jnp.bfloat16)
    a = jnp.dot(h0, w2_ref[:128, :], preferred_element_type=jnp.float32)
    b = jnp.dot(h1, w2_ref[128:, :], preferred_element_type=jnp.float32)
    o_ref[...] = a + b


NCLS = NCLASS


def _mm2(h_all, b1a, b1b, W2):
    return pl.pallas_call(
        _mm2_body,
        grid=(N_ROW_BLK,),
        in_specs=[
            pl.BlockSpec((ROW_BLK, 128), lambda i: (i, 0)),
            pl.BlockSpec((ROW_BLK, 128), lambda i: (N_ROW_BLK + i, 0)),
            pl.BlockSpec((8, 128), lambda i: (0, 0)),
            pl.BlockSpec((8, 128), lambda i: (0, 0)),
            pl.BlockSpec((NHID, NCLS), lambda i: (0, 0)),
        ],
        out_specs=pl.BlockSpec((ROW_BLK, NCLS), lambda i: (i, 0)),
        out_shape=jax.ShapeDtypeStruct((N_NODES, NCLS), jnp.float32),
    )(h_all, h_all, b1a, b1b, W2)


@functools.lru_cache(maxsize=None)
def _spmm2_kernel():
    @functools.partial(
        pl.kernel,
        out_type=jax.ShapeDtypeStruct((NC * N_NODES, NCLS), jnp.float32),
        mesh=_mesh(),
        scratch_types=[
            pltpu.VMEM((IB, CHW2), jnp.int32),
            pltpu.VMEM((IB, CHW2), jnp.int32),
            [pltpu.VMEM((CHW2, NCLS), jnp.float32) for _ in range(D2)],
            pltpu.VMEM_SHARED((N_NODES, NCLS), jnp.float32),
            [pltpu.SemaphoreType.DMA for _ in range(D2)],
        ],
        compiler_params=pltpu.CompilerParams(use_tc_tiling_on_sc=False),
    )
    def spmm2(s2_hbm, src_hbm, dst_hbm, zeros_hbm, p_hbm,
              isrc_v, idst_v, bufs, acc, sems):
        c = lax.axis_index("c")
        s = lax.axis_index("s")
        wid = c * NS + s

        _zero_acc(zeros_hbm, bufs[0].at[pl.ds(0, DRCH)], acc, s)
        plsc.subcore_barrier()

        def gather(j, k):
            pltpu.async_copy(s2_hbm.at[isrc_v.at[j]], bufs[k], sems[k])

        def wait_gather(k):
            pltpu.make_async_copy(
                s2_hbm.at[isrc_v.at[0]], bufs[k], sems[k]).wait()

        def block_body(b, carry):
            pltpu.sync_copy(src_hbm.at[wid, b], isrc_v)
            pltpu.sync_copy(dst_hbm.at[wid, b], idst_v)
            for k in range(D2 - 1):
                gather(k, k)
            for j in range(IB):
                wait_gather(j % D2)
                if j + D2 - 1 < IB:
                    gather(j + D2 - 1, (j + D2 - 1) % D2)
                pltpu.sync_copy(bufs[j % D2], acc.at[idst_v.at[j]], add=True)
            return carry

        lax.fori_loop(0, NB2, block_body, 0)

        plsc.subcore_barrier()

        for i in range(DR_PER_TILE):
            ch = s + NS * i

            @pl.when(ch < NDRCH)
            def _():
                r0 = ch * DRCH
                pltpu.sync_copy(acc.at[pl.ds(r0, DRCH)], bufs[0].at[pl.ds(0, DRCH)])
                pltpu.sync_copy(bufs[0].at[pl.ds(0, DRCH)],
                                p_hbm.at[pl.ds(c * N_NODES + r0, DRCH)])

    return spmm2


def _comb_body(p0_ref, p1_ref, b2_ref, o_ref):
    o_ref[...] = p0_ref[...] + p1_ref[...] + b2_ref[0:1, :]


def _comb(p_all, b2):
    return pl.pallas_call(
        _comb_body,
        grid=(N_ROW_BLK,),
        in_specs=[
            pl.BlockSpec((ROW_BLK, NCLS), lambda i: (i, 0)),
            pl.BlockSpec((ROW_BLK, NCLS), lambda i: (N_ROW_BLK + i, 0)),
            pl.BlockSpec((8, NCLASS), lambda i: (0, 0)),
        ],
        out_specs=pl.BlockSpec((ROW_BLK, NCLASS), lambda i: (i, 0)),
        out_shape=jax.ShapeDtypeStruct((N_NODES, NCLASS), jnp.float32),
    )(p_all, p_all, b2)


@jax.jit
def kernel(x, adj, W1, b1, W2, b2):
    src = adj[0].astype(jnp.int32)
    dst = adj[1].astype(jnp.int32)
    src1 = src.reshape(NS, NB1, IB, CHW1)
    dst1 = dst.reshape(NS, NB1, IB, CHW1)
    src2 = src.reshape(NC * NS, NB2, IB, CHW2)
    dst2 = dst.reshape(NC * NS, NB2, IB, CHW2)
    zeros1 = jnp.zeros((DRCH, 128), jnp.float32)
    zeros2 = jnp.zeros((DRCH, NCLASS), jnp.float32)
    b1a = jnp.broadcast_to(b1[:128].reshape(1, 128), (8, 128))
    b1b = jnp.broadcast_to(b1[128:].reshape(1, 128), (8, 128))
    b2r = jnp.broadcast_to(b2.reshape(1, NCLASS), (8, NCLASS))

    sup = _mm1(x.astype(jnp.bfloat16), W1.astype(jnp.bfloat16))
    h_all = _spmm1_kernel()(sup, src1, dst1, zeros1)
    s2 = _mm2(h_all, b1a, b1b, W2.astype(jnp.bfloat16))
    p_all = _spmm2_kernel()(s2, src2, dst2, zeros2)
    return _comb(p_all, b2r)


# speedup vs baseline: 1.1185x; 1.0220x over previous
import functools

import jax
import jax.numpy as jnp
from jax import lax
from jax.experimental import pallas as pl
from jax.experimental.pallas import tpu as pltpu
from jax.experimental.pallas import tpu_sc as plsc

N_NODES = 10000
N_EDGES = 160000
NFEAT = 256
NHID = 256
NCLASS = 64
NCLS_P = 128

NC = 2
NS = 16

ROW_BLK = 1000
N_ROW_BLK = N_NODES // ROW_BLK

DRCH = 80
NDRCH = N_NODES // DRCH
DR_PER_TILE = -(-NDRCH // NS)

CHW1 = 80
CHW2 = 100
IB = 25
NB1 = N_EDGES // NS // (IB * CHW1)
NB2 = N_EDGES // (NC * NS) // (IB * CHW2)
D1 = 4
D2 = 4


@functools.lru_cache(maxsize=None)
def _mesh():
    return plsc.VectorSubcoreMesh(
        core_axis_name="c", subcore_axis_name="s", num_cores=NC, num_subcores=NS
    )


def _mm1_body(x_ref, w_ref, o_ref):
    o_ref[...] = jnp.dot(x_ref[...], w_ref[...],
                         preferred_element_type=jnp.float32)


def _mm1(x, W1):
    return pl.pallas_call(
        _mm1_body,
        grid=(NC, N_ROW_BLK),
        in_specs=[
            pl.BlockSpec((ROW_BLK, NFEAT), lambda h, i: (i, 0)),
            pl.BlockSpec((NFEAT, 128), lambda h, i: (0, h)),
        ],
        out_specs=pl.BlockSpec((ROW_BLK, 128),
                               lambda h, i: (h * N_ROW_BLK + i, 0)),
        out_shape=jax.ShapeDtypeStruct((NC * N_NODES, 128), jnp.float32),
    )(x, W1)


def _zero_acc(zeros_hbm, buf_v, acc, s):
    pltpu.sync_copy(zeros_hbm, buf_v)
    for i in range(DR_PER_TILE):
        ch = s + NS * i

        @pl.when(ch < NDRCH)
        def _():
            pltpu.sync_copy(buf_v, acc.at[pl.ds(ch * DRCH, DRCH)])


@functools.lru_cache(maxsize=None)
def _spmm1_kernel():
    @functools.partial(
        pl.kernel,
        out_type=jax.ShapeDtypeStruct((NC * N_NODES, 128), jnp.float32),
        mesh=_mesh(),
        scratch_types=[
            pltpu.VMEM((IB, CHW1), jnp.int32),
            pltpu.VMEM((IB, CHW1), jnp.int32),
            [pltpu.VMEM((CHW1, 128), jnp.float32) for _ in range(D1)],
            pltpu.VMEM_SHARED((N_NODES, 128), jnp.float32),
            [pltpu.SemaphoreType.DMA for _ in range(D1)],
        ],
    )
    def spmm1(sup_hbm, src_hbm, dst_hbm, zeros_hbm, h_hbm,
              isrc_v, idst_v, bufs, acc, sems):
        c = lax.axis_index("c")
        s = lax.axis_index("s")

        pltpu.sync_copy(src_hbm.at[s, 0], isrc_v)
        pltpu.sync_copy(dst_hbm.at[s, 0], idst_v)
        _zero_acc(zeros_hbm, bufs[0], acc, s)
        plsc.subcore_barrier()

        half = sup_hbm.at[pl.ds(c * N_NODES, N_NODES)]

        def gather(j, k):
            pltpu.async_copy(half.at[isrc_v.at[j]], bufs[k], sems[k])

        def wait_gather(k):
            pltpu.make_async_copy(
                half.at[isrc_v.at[0]], bufs[k], sems[k]).wait()

        def block_body(b, carry):
            @pl.when(b > 0)
            def _():
                pltpu.sync_copy(src_hbm.at[s, b], isrc_v)
                pltpu.sync_copy(dst_hbm.at[s, b], idst_v)
            for k in range(D1 - 1):
                gather(k, k)
            for j in range(IB):
                wait_gather(j % D1)
                if j + D1 - 1 < IB:
                    gather(j + D1 - 1, (j + D1 - 1) % D1)
                pltpu.sync_copy(bufs[j % D1], acc.at[idst_v.at[j]], add=True)
            return carry

        lax.fori_loop(0, NB1, block_body, 0)

        plsc.subcore_barrier()

        for i in range(DR_PER_TILE):
            ch = s + NS * i

            @pl.when(ch < NDRCH)
            def _():
                r0 = ch * DRCH
                pltpu.sync_copy(acc.at[pl.ds(r0, DRCH)], bufs[0])
                pltpu.sync_copy(bufs[0],
                                h_hbm.at[pl.ds(c * N_NODES + r0, DRCH)])

    return spmm1


def _mm2_body(h0_ref, h1_ref, b1a_ref, b1b_ref, w2_ref, o_ref):
    h0 = jnp.maximum(h0_ref[...] + b1a_ref[0:1, :], 0.0)
    h1 = jnp.maximum(h1_ref[...] + b1b_ref[0:1, :], 0.0)
    a = jnp.dot(h0, w2_ref[:128, :], preferred_element_type=jnp.float32)
    b = jnp.dot(h1, w2_ref[128:, :], preferred_element_type=jnp.float32)
    o_ref[...] = a + b


NCLS = NCLASS


def _mm2(h_all, b1a, b1b, W2):
    return pl.pallas_call(
        _mm2_body,
        grid=(N_ROW_BLK,),
        in_specs=[
            pl.BlockSpec((ROW_BLK, 128), lambda i: (i, 0)),
            pl.BlockSpec((ROW_BLK, 128), lambda i: (N_ROW_BLK + i, 0)),
            pl.BlockSpec((8, 128), lambda i: (0, 0)),
            pl.BlockSpec((8, 128), lambda i: (0, 0)),
            pl.BlockSpec((NHID, NCLS), lambda i: (0, 0)),
        ],
        out_specs=pl.BlockSpec((ROW_BLK, NCLS), lambda i: (i, 0)),
        out_shape=jax.ShapeDtypeStruct((N_NODES, NCLS), jnp.float32),
    )(h_all, h_all, b1a, b1b, W2)


@functools.lru_cache(maxsize=None)
def _spmm2_kernel():
    @functools.partial(
        pl.kernel,
        out_type=jax.ShapeDtypeStruct((NC * N_NODES, NCLS), jnp.float32),
        mesh=_mesh(),
        scratch_types=[
            pltpu.VMEM((IB, CHW2), jnp.int32),
            pltpu.VMEM((IB, CHW2), jnp.int32),
            [pltpu.VMEM((CHW2, NCLS), jnp.float32) for _ in range(D2)],
            pltpu.VMEM_SHARED((N_NODES, NCLS), jnp.float32),
            [pltpu.SemaphoreType.DMA for _ in range(D2)],
        ],
        compiler_params=pltpu.CompilerParams(use_tc_tiling_on_sc=False),
    )
    def spmm2(s2_hbm, src_hbm, dst_hbm, zeros_hbm, p_hbm,
              isrc_v, idst_v, bufs, acc, sems):
        c = lax.axis_index("c")
        s = lax.axis_index("s")
        wid = c * NS + s

        pltpu.sync_copy(src_hbm.at[wid, 0], isrc_v)
        pltpu.sync_copy(dst_hbm.at[wid, 0], idst_v)
        _zero_acc(zeros_hbm, bufs[0].at[pl.ds(0, DRCH)], acc, s)
        plsc.subcore_barrier()

        def gather(j, k):
            pltpu.async_copy(s2_hbm.at[isrc_v.at[j]], bufs[k], sems[k])

        def wait_gather(k):
            pltpu.make_async_copy(
                s2_hbm.at[isrc_v.at[0]], bufs[k], sems[k]).wait()

        def block_body(b, carry):
            @pl.when(b > 0)
            def _():
                pltpu.sync_copy(src_hbm.at[wid, b], isrc_v)
                pltpu.sync_copy(dst_hbm.at[wid, b], idst_v)
            for k in range(D2 - 1):
                gather(k, k)
            for j in range(IB):
                wait_gather(j % D2)
                if j + D2 - 1 < IB:
                    gather(j + D2 - 1, (j + D2 - 1) % D2)
                pltpu.sync_copy(bufs[j % D2], acc.at[idst_v.at[j]], add=True)
            return carry

        lax.fori_loop(0, NB2, block_body, 0)

        plsc.subcore_barrier()

        for i in range(DR_PER_TILE):
            ch = s + NS * i

            @pl.when(ch < NDRCH)
            def _():
                r0 = ch * DRCH
                pltpu.sync_copy(acc.at[pl.ds(r0, DRCH)], bufs[0].at[pl.ds(0, DRCH)])
                pltpu.sync_copy(bufs[0].at[pl.ds(0, DRCH)],
                                p_hbm.at[pl.ds(c * N_NODES + r0, DRCH)])

    return spmm2


def _comb_body(p0_ref, p1_ref, b2_ref, o_ref):
    o_ref[...] = p0_ref[...] + p1_ref[...] + b2_ref[0:1, :]


def _comb(p_all, b2):
    return pl.pallas_call(
        _comb_body,
        grid=(N_ROW_BLK,),
        in_specs=[
            pl.BlockSpec((ROW_BLK, NCLS), lambda i: (i, 0)),
            pl.BlockSpec((ROW_BLK, NCLS), lambda i: (N_ROW_BLK + i, 0)),
            pl.BlockSpec((8, NCLASS), lambda i: (0, 0)),
        ],
        out_specs=pl.BlockSpec((ROW_BLK, NCLASS), lambda i: (i, 0)),
        out_shape=jax.ShapeDtypeStruct((N_NODES, NCLASS), jnp.float32),
    )(p_all, p_all, b2)


@jax.jit
def kernel(x, adj, W1, b1, W2, b2):
    src = adj[0].astype(jnp.int32)
    dst = adj[1].astype(jnp.int32)
    src1 = src.reshape(NS, NB1, IB, CHW1)
    dst1 = dst.reshape(NS, NB1, IB, CHW1)
    src2 = src.reshape(NC * NS, NB2, IB, CHW2)
    dst2 = dst.reshape(NC * NS, NB2, IB, CHW2)
    zeros1 = jnp.zeros((DRCH, 128), jnp.float32)
    zeros2 = jnp.zeros((DRCH, NCLASS), jnp.float32)
    b1a = jnp.broadcast_to(b1[:128].reshape(1, 128), (8, 128))
    b1b = jnp.broadcast_to(b1[128:].reshape(1, 128), (8, 128))
    b2r = jnp.broadcast_to(b2.reshape(1, NCLASS), (8, NCLASS))

    sup = _mm1(x, W1)
    h_all = _spmm1_kernel()(sup, src1, dst1, zeros1)
    s2 = _mm2(h_all, b1a, b1b, W2)
    p_all = _spmm2_kernel()(s2, src2, dst2, zeros2)
    return _comb(p_all, b2r)
